# baseline (speedup 1.0000x reference)
import functools

import jax
import jax.numpy as jnp
from jax import lax
from jax.experimental import pallas as pl
from jax.experimental.pallas import tpu as pltpu
from jax.experimental.pallas import tpu_sc as plsc

N = 50000
E = 800000
F = 64
D = 41
NCONV = 5
LATENT = 128
NCH = 3
NGRAPH = 128

NS = 16
HF = 32
CHUNK = 48
NPAD = 50240
EPAD = 801792
EPT = EPAD // NS
NCHK = EPT // CHUNK
G = 36
NGRP = NCHK // G
HALF = EPAD // 2
EAB2 = 384
RPT = NPAD // NS
TCB = 1256
NBLK = NPAD // TCB
EAB = 1536

_LOG1P = (1.1447097560735031e-05, 0.9991664010110692, -0.48969909032083947,
          0.28382318306531834, -0.1299571976582333, 0.029808765243435193)


def _embed_body(x_ref, w_ref, b_ref, o_ref):
    o_ref[...] = jnp.dot(x_ref[...], w_ref[...],
                         preferred_element_type=jnp.float32) + b_ref[...]


def _ea_body(eaA_ref, eaB_ref, w_ref, b_ref, o_ref):
    A = jnp.dot(eaA_ref[...], w_ref[...],
                preferred_element_type=jnp.float32) + b_ref[...]
    B = jnp.dot(eaB_ref[...], w_ref[...],
                preferred_element_type=jnp.float32) + b_ref[...]
    for j in range(2 * NCONV):
        o_ref[j] = jnp.concatenate(
            [A[:, 64 * j:64 * j + 64], B[:, 64 * j:64 * j + 64]], axis=1)


def _tab_body(h_ref, w_ref, t0, t1, t2, t3):
    h = h_ref[...]
    for k, t in enumerate((t0, t1, t2, t3)):
        t[...] = jnp.dot(h, w_ref[k], preferred_element_type=jnp.float32)


def _stats_body(a_ref, s_ref, q_ref):
    pid = pl.program_id(0)

    @pl.when(pid == 0)
    def _():
        s_ref[...] = jnp.zeros_like(s_ref)
        q_ref[...] = jnp.zeros_like(q_ref)

    a = a_ref[...]
    ri = lax.broadcasted_iota(jnp.int32, a.shape, 1) + pid * TCB
    a = jnp.where(ri < N, a, 0.0)
    s_ref[...] += jnp.sum(a, axis=1)
    q_ref[...] += jnp.sum(a * a, axis=1)


def _apply_body(a_ref, h_ref, s_ref, q_ref, g_ref, b_ref, w_ref,
                hn_ref, t0, t1, t2, t3):
    a = jnp.concatenate([a_ref[0], a_ref[1]], axis=1)
    mean2 = s_ref[...] * (1.0 / N)
    var2 = q_ref[...] * (1.0 / N) - mean2 * mean2
    rs2 = lax.rsqrt(var2 + 1e-5)
    mean = jnp.concatenate([mean2[0:1, :], mean2[1:2, :]], axis=1)
    rs = jnp.concatenate([rs2[0:1, :], rs2[1:2, :]], axis=1)
    hn = (a - mean) * rs * g_ref[...] + b_ref[...] + h_ref[...]
    hn_ref[...] = hn
    for k, t in enumerate((t0, t1, t2, t3)):
        t[...] = jnp.dot(hn, w_ref[k], preferred_element_type=jnp.float32)


def _pool_body(h_ref, bt_ref, s1_ref, s2_ref):
    pid = pl.program_id(0)

    @pl.when(pid == 0)
    def _():
        s1_ref[...] = jnp.zeros_like(s1_ref)
        s2_ref[...] = jnp.zeros_like(s2_ref)

    h = h_ref[...]
    b = bt_ref[0, 0, :]
    gi = lax.broadcasted_iota(jnp.int32, (TCB, NGRAPH), 1)
    oh = jnp.broadcast_to(b[:, None], (TCB, NGRAPH)) == gi
    rows = lax.broadcasted_iota(jnp.int32, (TCB, NGRAPH), 0) + pid * TCB
    ohf = jnp.where(oh & (rows < N), 1.0, 0.0)
    dn = (((0,), (0,)), ((), ()))
    s1_ref[...] += lax.dot_general(ohf, h, dn,
                                   preferred_element_type=jnp.float32)
    s2_ref[...] += lax.dot_general(ohf, jnp.ones_like(h), dn,
                                   preferred_element_type=jnp.float32)


def _softplus_tc(x):
    return jnp.maximum(x, 0.0) + jnp.log(1.0 + jnp.exp(-jnp.abs(x)))


def _head_body(s1_ref, s2_ref, w1, b1, w2, b2, w3, b3, o_ref):
    pooled = s1_ref[...] / jnp.maximum(s2_ref[...], 1.0)
    a = _softplus_tc(jnp.dot(pooled, w1[...],
                             preferred_element_type=jnp.float32) + b1[...])
    a = _softplus_tc(jnp.dot(a, w2[...],
                             preferred_element_type=jnp.float32) + b2[...])
    o_ref[...] = jnp.dot(a, w3[...],
                         preferred_element_type=jnp.float32) + b3[...]


def _sigmoid_sc(x):
    return 1.0 / (1.0 + jnp.exp(-x))


def _softplus_sc(x):
    t = jnp.exp(-jnp.abs(x))
    c0, c1, c2, c3, c4, c5 = _LOG1P
    t2 = t * t
    p01 = c0 + c1 * t
    p23 = c2 + c3 * t
    p45 = c4 + c5 * t
    return jnp.maximum(x, 0.0) + (p01 + t2 * (p23 + t2 * p45))


def _make_sc_conv(layer):
    mesh = plsc.VectorSubcoreMesh(core_axis_name="c", subcore_axis_name="s",
                                  num_cores=2, num_subcores=NS)

    @functools.partial(
        pl.kernel, mesh=mesh,
        compiler_params=pltpu.CompilerParams(use_tc_tiling_on_sc=False),
        out_type=jax.ShapeDtypeStruct((2, NPAD, HF), jnp.float32),
        scratch_types=[
            pltpu.VMEM_SHARED((NPAD, HF), jnp.float32),
            pltpu.VMEM((2, G * CHUNK), jnp.int32),
            pltpu.VMEM((2, G * CHUNK), jnp.int32),
            pltpu.VMEM((3, CHUNK), jnp.int32),
            pltpu.VMEM((3, CHUNK), jnp.int32),
            pltpu.VMEM((2, CHUNK, 2 * HF), jnp.float32),
            pltpu.VMEM((2, CHUNK, 2 * HF), jnp.float32),
            pltpu.VMEM((2, CHUNK, 2 * HF), jnp.float32),
            pltpu.VMEM((2, CHUNK, HF), jnp.float32),
            pltpu.SemaphoreType.DMA,
            pltpu.SemaphoreType.DMA,
            pltpu.SemaphoreType.DMA,
        ],
    )
    def conv(td0, td1, tu0, tu1, eat, didx, sidx, zrs, out,
             accum, idxd1, idxs1, vd, vs, rT, rU, eb, mg,
             semA, semB, semS):
        c = lax.axis_index("c")
        s = lax.axis_index("s")
        erow = lax.rem(s, 8) * EPT
        coff = lax.div(s, 8) * 64

        pltpu.sync_copy(zrs.at[pl.ds(s * RPT, RPT)],
                        accum.at[pl.ds(s * RPT, RPT)])
        plsc.subcore_barrier()

        def load_group(g):
            gsl = lax.rem(g, 2)
            e0 = s * EPT + g * (G * CHUNK)
            pltpu.sync_copy(didx.at[pl.ds(e0, G * CHUNK)], idxd1.at[gsl])
            pltpu.sync_copy(sidx.at[pl.ds(e0, G * CHUNK)], idxs1.at[gsl])

        def prep(k1):
            sl3 = lax.rem(k1, 3)
            gsl = lax.rem(lax.div(k1, G), 2)
            j0 = lax.rem(k1, G) * CHUNK
            for q in range(CHUNK // 16):
                vd[sl3, pl.ds(q * 16, 16)] = idxd1[gsl, pl.ds(j0 + q * 16, 16)]
                vs[sl3, pl.ds(q * 16, 16)] = idxs1[gsl, pl.ds(j0 + q * 16, 16)]

        def issue(k1):
            sl = lax.rem(k1, 2)
            sl3 = lax.rem(k1, 3)
            rb = erow + k1 * CHUNK
            jl0 = 2 * layer
            jl1 = 2 * layer + 1

            @pl.when(sl == 0)
            def _():
                @pl.when(c == 0)
                def _():
                    pltpu.async_copy(td0.at[vd.at[sl3]], rT.at[0], semA)
                    pltpu.async_copy(tu0.at[vs.at[sl3]], rU.at[0], semA)
                    pltpu.async_copy(
                        eat.at[jl0, pl.ds(rb, CHUNK), pl.ds(coff, 64)],
                        eb.at[0], semA)

                @pl.when(c == 1)
                def _():
                    pltpu.async_copy(td1.at[vd.at[sl3]], rT.at[0], semA)
                    pltpu.async_copy(tu1.at[vs.at[sl3]], rU.at[0], semA)
                    pltpu.async_copy(
                        eat.at[jl1, pl.ds(rb, CHUNK), pl.ds(coff, 64)],
                        eb.at[0], semA)

            @pl.when(sl == 1)
            def _():
                @pl.when(c == 0)
                def _():
                    pltpu.async_copy(td0.at[vd.at[sl3]], rT.at[1], semB)
                    pltpu.async_copy(tu0.at[vs.at[sl3]], rU.at[1], semB)
                    pltpu.async_copy(
                        eat.at[jl0, pl.ds(rb, CHUNK), pl.ds(coff, 64)],
                        eb.at[1], semB)

                @pl.when(c == 1)
                def _():
                    pltpu.async_copy(td1.at[vd.at[sl3]], rT.at[1], semB)
                    pltpu.async_copy(tu1.at[vs.at[sl3]], rU.at[1], semB)
                    pltpu.async_copy(
                        eat.at[jl1, pl.ds(rb, CHUNK), pl.ds(coff, 64)],
                        eb.at[1], semB)

        def drain_gathers(sl):
            @pl.when(sl == 0)
            def _():
                pltpu.make_async_copy(td0.at[pl.ds(0, CHUNK)], rT.at[0], semA).wait()
                pltpu.make_async_copy(td0.at[pl.ds(0, CHUNK)], rU.at[0], semA).wait()
                pltpu.make_async_copy(td0.at[pl.ds(0, CHUNK)], eb.at[0], semA).wait()

            @pl.when(sl == 1)
            def _():
                pltpu.make_async_copy(td0.at[pl.ds(0, CHUNK)], rT.at[1], semB).wait()
                pltpu.make_async_copy(td0.at[pl.ds(0, CHUNK)], rU.at[1], semB).wait()
                pltpu.make_async_copy(td0.at[pl.ds(0, CHUNK)], eb.at[1], semB).wait()

        def drain_scatter():
            pltpu.make_async_copy(mg.at[0], accum.at[pl.ds(0, CHUNK)],
                                  semS).wait()

        load_group(0)
        prep(0)
        issue(0)

        def step(k, _):
            k1 = k + 1

            @pl.when(k >= 2)
            def _():
                drain_scatter()

            @pl.when(k1 < NCHK)
            def _():
                @pl.when(lax.rem(k1, G) == 0)
                def _():
                    load_group(lax.div(k1, G))

                prep(k1)
                issue(k1)

            sl = lax.rem(k, 2)
            sl3 = lax.rem(k, 3)
            drain_gathers(sl)

            def edge(p, _):
                for u in range(4):
                    e = 4 * p + u
                    for q in range(2):
                        fo, so = q * 16, HF + q * 16
                        pf = (rT[sl, e, pl.ds(fo, 16)]
                              + rU[sl, e, pl.ds(fo, 16)]
                              + eb[sl, e, pl.ds(fo, 16)])
                        ps = (rT[sl, e, pl.ds(so, 16)]
                              + rU[sl, e, pl.ds(so, 16)]
                              + eb[sl, e, pl.ds(so, 16)])
                        mg[sl, e, pl.ds(fo, 16)] = (_sigmoid_sc(pf)
                                                    * _softplus_sc(ps))
                return 0

            lax.fori_loop(0, CHUNK // 4, edge, 0)
            pltpu.async_copy(mg.at[sl], accum.at[vd.at[sl3]], semS, add=True)
            return 0

        lax.fori_loop(0, NCHK, step, 0)
        drain_scatter()
        drain_scatter()
        plsc.subcore_barrier()

        @pl.when(c == 0)
        def _():
            pltpu.sync_copy(accum.at[pl.ds(s * RPT, RPT)],
                            out.at[0, pl.ds(s * RPT, RPT)])

        @pl.when(c == 1)
        def _():
            pltpu.sync_copy(accum.at[pl.ds(s * RPT, RPT)],
                            out.at[1, pl.ds(s * RPT, RPT)])

    return conv


def kernel(x, edge_index, edge_attr, batch, emb_w, emb_b, lf_w, lf_b,
           ls_w, ls_b, bn_g, bn_b, h1_w, h1_b, h2_w, h2_b, h3_w, h3_b):
    f32 = jnp.float32

    x_p = jnp.zeros((NPAD, 128), f32).at[:N, :92].set(x)
    ew_p = jnp.zeros((128, F), f32).at[:92].set(emb_w)
    eb_r = emb_b.reshape(1, F)

    ea_p = jnp.zeros((EPAD, F), f32).at[:E, :D].set(edge_attr)

    web = []
    beb = []
    for l in range(NCONV):
        for c in range(2):
            wf = lf_w[l, 2 * F:, c * HF:(c + 1) * HF]
            ws = ls_w[l, 2 * F:, c * HF:(c + 1) * HF]
            web.append(jnp.concatenate([wf, ws], axis=1))
            beb.append(jnp.concatenate([lf_b[l, c * HF:(c + 1) * HF],
                                        ls_b[l, c * HF:(c + 1) * HF]]))
    web = jnp.concatenate(
        [jnp.pad(w, ((0, F - D), (0, 0))) for w in web], axis=1)
    beb = jnp.concatenate(beb).reshape(1, 2 * NCONV * F)

    wtab = []
    for l in range(NCONV):
        per = []
        for tbl in range(2):
            r0 = tbl * F
            for c in range(2):
                per.append(jnp.concatenate(
                    [lf_w[l, r0:r0 + F, c * HF:(c + 1) * HF],
                     ls_w[l, r0:r0 + F, c * HF:(c + 1) * HF]], axis=1))
        wtab.append(jnp.stack(per))
    wtab = jnp.stack(wtab)

    dpad = jnp.full((EPAD,), N, jnp.int32).at[:E].set(edge_index[1])
    spad = jnp.full((EPAD,), N, jnp.int32).at[:E].set(edge_index[0])
    zrs = jnp.zeros((NPAD, HF), jnp.float32)
    bt3 = jnp.zeros((NBLK, 1, TCB), jnp.int32).at[:, 0, :].set(
        jnp.pad(batch, (0, NPAD - N)).reshape(NBLK, TCB)[:, :])

    h = pl.pallas_call(
        _embed_body,
        grid=(NBLK,),
        in_specs=[pl.BlockSpec((TCB, 128), lambda i: (i, 0)),
                  pl.BlockSpec((128, F), lambda i: (0, 0)),
                  pl.BlockSpec((1, F), lambda i: (0, 0))],
        out_specs=pl.BlockSpec((TCB, F), lambda i: (i, 0)),
        out_shape=jax.ShapeDtypeStruct((NPAD, F), f32),
    )(x_p, ew_p, eb_r)

    nb = HALF // EAB2
    eat = pl.pallas_call(
        _ea_body,
        grid=(nb,),
        in_specs=[pl.BlockSpec((EAB2, F), lambda e: (e, 0)),
                  pl.BlockSpec((EAB2, F), lambda e, _nb=nb: (e + _nb, 0)),
                  pl.BlockSpec((F, 2 * NCONV * F), lambda e: (0, 0)),
                  pl.BlockSpec((1, 2 * NCONV * F), lambda e: (0, 0))],
        out_specs=pl.BlockSpec((2 * NCONV, EAB2, 2 * F), lambda e: (0, e, 0)),
        out_shape=jax.ShapeDtypeStruct((2 * NCONV, HALF, 2 * F), f32),
    )(ea_p, ea_p, web, beb)

    tab_specs = dict(
        grid=(NBLK,),
        out_specs=[pl.BlockSpec((TCB, F), lambda i: (i, 0))] * 4,
        out_shape=[jax.ShapeDtypeStruct((NPAD, F), f32)] * 4,
    )
    td0, td1, tu0, tu1 = pl.pallas_call(
        _tab_body,
        in_specs=[pl.BlockSpec((TCB, F), lambda i: (i, 0)),
                  pl.BlockSpec((4, F, F), lambda i: (0, 0, 0))],
        **tab_specs,
    )(h, wtab[0])

    for l in range(NCONV):
        agg = _make_sc_conv(l)(td0, td1, tu0, tu1, eat, dpad, spad, zrs)

        sums, sumsq = pl.pallas_call(
            _stats_body,
            grid=(NBLK,),
            in_specs=[pl.BlockSpec((2, TCB, HF), lambda i: (0, i, 0))],
            out_specs=[pl.BlockSpec((2, HF), lambda i: (0, 0))] * 2,
            out_shape=[jax.ShapeDtypeStruct((2, HF), f32)] * 2,
        )(agg)

        wnext = wtab[l + 1] if l + 1 < NCONV else wtab[0]
        h, td0, td1, tu0, tu1 = pl.pallas_call(
            _apply_body,
            grid=(NBLK,),
            in_specs=[pl.BlockSpec((2, TCB, HF), lambda i: (0, i, 0)),
                      pl.BlockSpec((TCB, F), lambda i: (i, 0)),
                      pl.BlockSpec((2, HF), lambda i: (0, 0)),
                      pl.BlockSpec((2, HF), lambda i: (0, 0)),
                      pl.BlockSpec((1, F), lambda i: (0, 0)),
                      pl.BlockSpec((1, F), lambda i: (0, 0)),
                      pl.BlockSpec((4, F, F), lambda i: (0, 0, 0))],
            out_specs=[pl.BlockSpec((TCB, F), lambda i: (i, 0))] * 5,
            out_shape=[jax.ShapeDtypeStruct((NPAD, F), f32)] * 5,
        )(agg, h, sums, sumsq, bn_g[l].reshape(1, F),
          bn_b[l].reshape(1, F), wnext)

    s1, s2 = pl.pallas_call(
        _pool_body,
        grid=(NBLK,),
        in_specs=[pl.BlockSpec((TCB, F), lambda i: (i, 0)),
                  pl.BlockSpec((1, 1, TCB), lambda i: (i, 0, 0))],
        out_specs=[pl.BlockSpec((NGRAPH, F), lambda i: (0, 0))] * 2,
        out_shape=[jax.ShapeDtypeStruct((NGRAPH, F), f32)] * 2,
    )(h, bt3)

    out = pl.pallas_call(
        _head_body,
        grid=(1,),
        in_specs=[pl.BlockSpec((NGRAPH, F), lambda i: (0, 0)),
                  pl.BlockSpec((NGRAPH, F), lambda i: (0, 0)),
                  pl.BlockSpec((F, 256), lambda i: (0, 0)),
                  pl.BlockSpec((1, 256), lambda i: (0, 0)),
                  pl.BlockSpec((256, 256), lambda i: (0, 0)),
                  pl.BlockSpec((1, 256), lambda i: (0, 0)),
                  pl.BlockSpec((256, LATENT * NCH), lambda i: (0, 0)),
                  pl.BlockSpec((1, LATENT * NCH), lambda i: (0, 0))],
        out_specs=pl.BlockSpec((NGRAPH, LATENT * NCH), lambda i: (0, 0)),
        out_shape=jax.ShapeDtypeStruct((NGRAPH, LATENT * NCH), f32),
    )(s1, s2, h1_w, h1_b.reshape(1, 256), h2_w, h2_b.reshape(1, 256),
      h3_w, h3_b.reshape(1, LATENT * NCH))

    return out.reshape(NGRAPH, NCH, LATENT)


# speedup vs baseline: 1.1865x; 1.1865x over previous
import functools

import jax
import jax.numpy as jnp
from jax import lax
from jax.experimental import pallas as pl
from jax.experimental.pallas import tpu as pltpu
from jax.experimental.pallas import tpu_sc as plsc

N = 50000
E = 800000
F = 64
D = 41
NCONV = 5
LATENT = 128
NCH = 3
NGRAPH = 128

NS = 16
HF = 32
CHUNK = 48
NPAD = 50240
EPAD = 801792
EPT = EPAD // NS
NCHK = EPT // CHUNK
G = 36
NGRP = NCHK // G
HALF = EPAD // 2
EAB2 = 384
RPT = NPAD // NS
TCB = 1256
NBLK = NPAD // TCB
EAB = 1536

_LOG1P = (1.1447097560735031e-05, 0.9991664010110692, -0.48969909032083947,
          0.28382318306531834, -0.1299571976582333, 0.029808765243435193)


def _embed_body(x_ref, w_ref, b_ref, o_ref):
    o_ref[...] = jnp.dot(x_ref[...], w_ref[...],
                         preferred_element_type=jnp.float32) + b_ref[...]


def _ea_body(eaA_ref, eaB_ref, w_ref, b_ref, o_ref):
    A = jnp.dot(eaA_ref[...], w_ref[...],
                preferred_element_type=jnp.float32) + b_ref[...]
    B = jnp.dot(eaB_ref[...], w_ref[...],
                preferred_element_type=jnp.float32) + b_ref[...]
    for j in range(2 * NCONV):
        o_ref[j] = jnp.concatenate(
            [A[:, 64 * j:64 * j + 64], B[:, 64 * j:64 * j + 64]], axis=1)


def _tab_body(h_ref, w_ref, t0, t1, t2, t3):
    h = h_ref[...]
    for k, t in enumerate((t0, t1, t2, t3)):
        t[...] = jnp.dot(h, w_ref[k], preferred_element_type=jnp.float32)


def _stats_body(a_ref, s_ref, q_ref):
    pid = pl.program_id(0)

    @pl.when(pid == 0)
    def _():
        s_ref[...] = jnp.zeros_like(s_ref)
        q_ref[...] = jnp.zeros_like(q_ref)

    a = a_ref[...]
    ri = lax.broadcasted_iota(jnp.int32, a.shape, 1) + pid * TCB
    a = jnp.where(ri < N, a, 0.0)
    s_ref[...] += jnp.sum(a, axis=1)
    q_ref[...] += jnp.sum(a * a, axis=1)


def _apply_body(a_ref, h_ref, s_ref, q_ref, g_ref, b_ref, w_ref,
                hn_ref, t0, t1, t2, t3):
    a = jnp.concatenate([a_ref[0], a_ref[1]], axis=1)
    mean2 = s_ref[...] * (1.0 / N)
    var2 = q_ref[...] * (1.0 / N) - mean2 * mean2
    rs2 = lax.rsqrt(var2 + 1e-5)
    mean = jnp.concatenate([mean2[0:1, :], mean2[1:2, :]], axis=1)
    rs = jnp.concatenate([rs2[0:1, :], rs2[1:2, :]], axis=1)
    hn = (a - mean) * rs * g_ref[...] + b_ref[...] + h_ref[...]
    hn_ref[...] = hn
    for k, t in enumerate((t0, t1, t2, t3)):
        t[...] = jnp.dot(hn, w_ref[k], preferred_element_type=jnp.float32)


def _pool_body(h_ref, bt_ref, s1_ref, s2_ref):
    pid = pl.program_id(0)

    @pl.when(pid == 0)
    def _():
        s1_ref[...] = jnp.zeros_like(s1_ref)
        s2_ref[...] = jnp.zeros_like(s2_ref)

    h = h_ref[...]
    b = bt_ref[0, 0, :]
    gi = lax.broadcasted_iota(jnp.int32, (TCB, NGRAPH), 1)
    oh = jnp.broadcast_to(b[:, None], (TCB, NGRAPH)) == gi
    rows = lax.broadcasted_iota(jnp.int32, (TCB, NGRAPH), 0) + pid * TCB
    ohf = jnp.where(oh & (rows < N), 1.0, 0.0)
    dn = (((0,), (0,)), ((), ()))
    s1_ref[...] += lax.dot_general(ohf, h, dn,
                                   preferred_element_type=jnp.float32)
    s2_ref[...] += lax.dot_general(ohf, jnp.ones_like(h), dn,
                                   preferred_element_type=jnp.float32)


def _softplus_tc(x):
    return jnp.maximum(x, 0.0) + jnp.log(1.0 + jnp.exp(-jnp.abs(x)))


def _head_body(s1_ref, s2_ref, w1, b1, w2, b2, w3, b3, o_ref):
    pooled = s1_ref[...] / jnp.maximum(s2_ref[...], 1.0)
    a = _softplus_tc(jnp.dot(pooled, w1[...],
                             preferred_element_type=jnp.float32) + b1[...])
    a = _softplus_tc(jnp.dot(a, w2[...],
                             preferred_element_type=jnp.float32) + b2[...])
    o_ref[...] = jnp.dot(a, w3[...],
                         preferred_element_type=jnp.float32) + b3[...]


def _sigmoid_sc(x):
    return 1.0 / (1.0 + jnp.exp(-x))


def _softplus_sc(x):
    t = jnp.exp(-jnp.abs(x))
    c0, c1, c2, c3, c4, c5 = _LOG1P
    t2 = t * t
    p01 = c0 + c1 * t
    p23 = c2 + c3 * t
    p45 = c4 + c5 * t
    return jnp.maximum(x, 0.0) + (p01 + t2 * (p23 + t2 * p45))


def _make_sc_conv(layer):
    mesh = plsc.VectorSubcoreMesh(core_axis_name="c", subcore_axis_name="s",
                                  num_cores=2, num_subcores=NS)

    @functools.partial(
        pl.kernel, mesh=mesh,
        compiler_params=pltpu.CompilerParams(use_tc_tiling_on_sc=False),
        out_type=jax.ShapeDtypeStruct((2, NPAD, HF), jnp.float32),
        scratch_types=[
            pltpu.VMEM_SHARED((NPAD, HF), jnp.float32),
            pltpu.VMEM((2, G * CHUNK), jnp.int32),
            pltpu.VMEM((2, G * CHUNK), jnp.int32),
            pltpu.VMEM((3, CHUNK), jnp.int32),
            pltpu.VMEM((3, CHUNK), jnp.int32),
            pltpu.VMEM((2, CHUNK, 2 * HF), jnp.float32),
            pltpu.VMEM((2, CHUNK, 2 * HF), jnp.float32),
            pltpu.VMEM((2, CHUNK, 2 * HF), jnp.float32),
            pltpu.VMEM((2, CHUNK, HF), jnp.float32),
            pltpu.SemaphoreType.DMA,
            pltpu.SemaphoreType.DMA,
            pltpu.SemaphoreType.DMA,
        ],
    )
    def conv(td0, td1, tu0, tu1, eat, didx, sidx, zrs, out,
             accum, idxd1, idxs1, vd, vs, rT, rU, eb, mg,
             semA, semB, semS):
        c = lax.axis_index("c")
        s = lax.axis_index("s")
        erow = lax.rem(s, 8) * EPT
        coff = lax.div(s, 8) * 64

        pltpu.sync_copy(zrs.at[pl.ds(s * RPT, RPT)],
                        accum.at[pl.ds(s * RPT, RPT)])
        plsc.subcore_barrier()

        def load_group(g):
            gsl = lax.rem(g, 2)
            e0 = s * EPT + g * (G * CHUNK)
            pltpu.sync_copy(didx.at[pl.ds(e0, G * CHUNK)], idxd1.at[gsl])
            pltpu.sync_copy(sidx.at[pl.ds(e0, G * CHUNK)], idxs1.at[gsl])

        def prep(k1):
            sl3 = lax.rem(k1, 3)
            gsl = lax.rem(lax.div(k1, G), 2)
            j0 = lax.rem(k1, G) * CHUNK
            for q in range(CHUNK // 16):
                vd[sl3, pl.ds(q * 16, 16)] = idxd1[gsl, pl.ds(j0 + q * 16, 16)]
                vs[sl3, pl.ds(q * 16, 16)] = idxs1[gsl, pl.ds(j0 + q * 16, 16)]

        def issue(k1):
            sl = lax.rem(k1, 2)
            sl3 = lax.rem(k1, 3)
            rb = erow + k1 * CHUNK
            jl0 = 2 * layer
            jl1 = 2 * layer + 1

            @pl.when(sl == 0)
            def _():
                @pl.when(c == 0)
                def _():
                    pltpu.async_copy(td0.at[vd.at[sl3]], rT.at[0], semA)
                    pltpu.async_copy(tu0.at[vs.at[sl3]], rU.at[0], semA)
                    pltpu.async_copy(
                        eat.at[jl0, pl.ds(rb, CHUNK), pl.ds(coff, 64)],
                        eb.at[0], semA)

                @pl.when(c == 1)
                def _():
                    pltpu.async_copy(td1.at[vd.at[sl3]], rT.at[0], semA)
                    pltpu.async_copy(tu1.at[vs.at[sl3]], rU.at[0], semA)
                    pltpu.async_copy(
                        eat.at[jl1, pl.ds(rb, CHUNK), pl.ds(coff, 64)],
                        eb.at[0], semA)

            @pl.when(sl == 1)
            def _():
                @pl.when(c == 0)
                def _():
                    pltpu.async_copy(td0.at[vd.at[sl3]], rT.at[1], semB)
                    pltpu.async_copy(tu0.at[vs.at[sl3]], rU.at[1], semB)
                    pltpu.async_copy(
                        eat.at[jl0, pl.ds(rb, CHUNK), pl.ds(coff, 64)],
                        eb.at[1], semB)

                @pl.when(c == 1)
                def _():
                    pltpu.async_copy(td1.at[vd.at[sl3]], rT.at[1], semB)
                    pltpu.async_copy(tu1.at[vs.at[sl3]], rU.at[1], semB)
                    pltpu.async_copy(
                        eat.at[jl1, pl.ds(rb, CHUNK), pl.ds(coff, 64)],
                        eb.at[1], semB)

        def drain_gathers(sl):
            @pl.when(sl == 0)
            def _():
                pltpu.make_async_copy(td0.at[pl.ds(0, CHUNK)], rT.at[0], semA).wait()
                pltpu.make_async_copy(td0.at[pl.ds(0, CHUNK)], rU.at[0], semA).wait()
                pltpu.make_async_copy(td0.at[pl.ds(0, CHUNK)], eb.at[0], semA).wait()

            @pl.when(sl == 1)
            def _():
                pltpu.make_async_copy(td0.at[pl.ds(0, CHUNK)], rT.at[1], semB).wait()
                pltpu.make_async_copy(td0.at[pl.ds(0, CHUNK)], rU.at[1], semB).wait()
                pltpu.make_async_copy(td0.at[pl.ds(0, CHUNK)], eb.at[1], semB).wait()

        def drain_scatter():
            pltpu.make_async_copy(mg.at[0], accum.at[pl.ds(0, CHUNK)],
                                  semS).wait()

        load_group(0)
        prep(0)
        issue(0)

        def step(k, _):
            k1 = k + 1

            @pl.when(k >= 2)
            def _():
                drain_scatter()

            @pl.when(k1 < NCHK)
            def _():
                @pl.when(lax.rem(k1, G) == 0)
                def _():
                    load_group(lax.div(k1, G))

                prep(k1)
                issue(k1)

            sl = lax.rem(k, 2)
            sl3 = lax.rem(k, 3)
            drain_gathers(sl)

            def edge(p, _):
                for u in range(2):
                    e = 2 * p + u
                    for q in range(2):
                        fo, so = q * 16, HF + q * 16
                        pf = (rT[sl, e, pl.ds(fo, 16)]
                              + rU[sl, e, pl.ds(fo, 16)]
                              + eb[sl, e, pl.ds(fo, 16)])
                        ps = (rT[sl, e, pl.ds(so, 16)]
                              + rU[sl, e, pl.ds(so, 16)]
                              + eb[sl, e, pl.ds(so, 16)])
                        mg[sl, e, pl.ds(fo, 16)] = (_sigmoid_sc(pf)
                                                    * _softplus_sc(ps))
                return 0

            lax.fori_loop(0, CHUNK // 2, edge, 0)
            pltpu.async_copy(mg.at[sl], accum.at[vd.at[sl3]], semS, add=True)
            return 0

        lax.fori_loop(0, NCHK, step, 0)
        drain_scatter()
        drain_scatter()
        plsc.subcore_barrier()

        @pl.when(c == 0)
        def _():
            pltpu.sync_copy(accum.at[pl.ds(s * RPT, RPT)],
                            out.at[0, pl.ds(s * RPT, RPT)])

        @pl.when(c == 1)
        def _():
            pltpu.sync_copy(accum.at[pl.ds(s * RPT, RPT)],
                            out.at[1, pl.ds(s * RPT, RPT)])

    return conv


def kernel(x, edge_index, edge_attr, batch, emb_w, emb_b, lf_w, lf_b,
           ls_w, ls_b, bn_g, bn_b, h1_w, h1_b, h2_w, h2_b, h3_w, h3_b):
    f32 = jnp.float32

    x_p = jnp.zeros((NPAD, 128), f32).at[:N, :92].set(x)
    ew_p = jnp.zeros((128, F), f32).at[:92].set(emb_w)
    eb_r = emb_b.reshape(1, F)

    ea_p = jnp.zeros((EPAD, F), f32).at[:E, :D].set(edge_attr)

    web = []
    beb = []
    for l in range(NCONV):
        for c in range(2):
            wf = lf_w[l, 2 * F:, c * HF:(c + 1) * HF]
            ws = ls_w[l, 2 * F:, c * HF:(c + 1) * HF]
            web.append(jnp.concatenate([wf, ws], axis=1))
            beb.append(jnp.concatenate([lf_b[l, c * HF:(c + 1) * HF],
                                        ls_b[l, c * HF:(c + 1) * HF]]))
    web = jnp.concatenate(
        [jnp.pad(w, ((0, F - D), (0, 0))) for w in web], axis=1)
    beb = jnp.concatenate(beb).reshape(1, 2 * NCONV * F)

    wtab = []
    for l in range(NCONV):
        per = []
        for tbl in range(2):
            r0 = tbl * F
            for c in range(2):
                per.append(jnp.concatenate(
                    [lf_w[l, r0:r0 + F, c * HF:(c + 1) * HF],
                     ls_w[l, r0:r0 + F, c * HF:(c + 1) * HF]], axis=1))
        wtab.append(jnp.stack(per))
    wtab = jnp.stack(wtab)

    dpad = jnp.full((EPAD,), N, jnp.int32).at[:E].set(edge_index[1])
    spad = jnp.full((EPAD,), N, jnp.int32).at[:E].set(edge_index[0])
    zrs = jnp.zeros((NPAD, HF), jnp.float32)
    bt3 = jnp.zeros((NBLK, 1, TCB), jnp.int32).at[:, 0, :].set(
        jnp.pad(batch, (0, NPAD - N)).reshape(NBLK, TCB)[:, :])

    h = pl.pallas_call(
        _embed_body,
        grid=(NBLK,),
        in_specs=[pl.BlockSpec((TCB, 128), lambda i: (i, 0)),
                  pl.BlockSpec((128, F), lambda i: (0, 0)),
                  pl.BlockSpec((1, F), lambda i: (0, 0))],
        out_specs=pl.BlockSpec((TCB, F), lambda i: (i, 0)),
        out_shape=jax.ShapeDtypeStruct((NPAD, F), f32),
    )(x_p, ew_p, eb_r)

    nb = HALF // EAB2
    eat = pl.pallas_call(
        _ea_body,
        grid=(nb,),
        in_specs=[pl.BlockSpec((EAB2, F), lambda e: (e, 0)),
                  pl.BlockSpec((EAB2, F), lambda e, _nb=nb: (e + _nb, 0)),
                  pl.BlockSpec((F, 2 * NCONV * F), lambda e: (0, 0)),
                  pl.BlockSpec((1, 2 * NCONV * F), lambda e: (0, 0))],
        out_specs=pl.BlockSpec((2 * NCONV, EAB2, 2 * F), lambda e: (0, e, 0)),
        out_shape=jax.ShapeDtypeStruct((2 * NCONV, HALF, 2 * F), f32),
    )(ea_p, ea_p, web, beb)

    tab_specs = dict(
        grid=(NBLK,),
        out_specs=[pl.BlockSpec((TCB, F), lambda i: (i, 0))] * 4,
        out_shape=[jax.ShapeDtypeStruct((NPAD, F), f32)] * 4,
    )
    td0, td1, tu0, tu1 = pl.pallas_call(
        _tab_body,
        in_specs=[pl.BlockSpec((TCB, F), lambda i: (i, 0)),
                  pl.BlockSpec((4, F, F), lambda i: (0, 0, 0))],
        **tab_specs,
    )(h, wtab[0])

    for l in range(NCONV):
        agg = _make_sc_conv(l)(td0, td1, tu0, tu1, eat, dpad, spad, zrs)

        sums, sumsq = pl.pallas_call(
            _stats_body,
            grid=(NBLK,),
            in_specs=[pl.BlockSpec((2, TCB, HF), lambda i: (0, i, 0))],
            out_specs=[pl.BlockSpec((2, HF), lambda i: (0, 0))] * 2,
            out_shape=[jax.ShapeDtypeStruct((2, HF), f32)] * 2,
        )(agg)

        wnext = wtab[l + 1] if l + 1 < NCONV else wtab[0]
        h, td0, td1, tu0, tu1 = pl.pallas_call(
            _apply_body,
            grid=(NBLK,),
            in_specs=[pl.BlockSpec((2, TCB, HF), lambda i: (0, i, 0)),
                      pl.BlockSpec((TCB, F), lambda i: (i, 0)),
                      pl.BlockSpec((2, HF), lambda i: (0, 0)),
                      pl.BlockSpec((2, HF), lambda i: (0, 0)),
                      pl.BlockSpec((1, F), lambda i: (0, 0)),
                      pl.BlockSpec((1, F), lambda i: (0, 0)),
                      pl.BlockSpec((4, F, F), lambda i: (0, 0, 0))],
            out_specs=[pl.BlockSpec((TCB, F), lambda i: (i, 0))] * 5,
            out_shape=[jax.ShapeDtypeStruct((NPAD, F), f32)] * 5,
        )(agg, h, sums, sumsq, bn_g[l].reshape(1, F),
          bn_b[l].reshape(1, F), wnext)

    s1, s2 = pl.pallas_call(
        _pool_body,
        grid=(NBLK,),
        in_specs=[pl.BlockSpec((TCB, F), lambda i: (i, 0)),
                  pl.BlockSpec((1, 1, TCB), lambda i: (i, 0, 0))],
        out_specs=[pl.BlockSpec((NGRAPH, F), lambda i: (0, 0))] * 2,
        out_shape=[jax.ShapeDtypeStruct((NGRAPH, F), f32)] * 2,
    )(h, bt3)

    out = pl.pallas_call(
        _head_body,
        grid=(1,),
        in_specs=[pl.BlockSpec((NGRAPH, F), lambda i: (0, 0)),
                  pl.BlockSpec((NGRAPH, F), lambda i: (0, 0)),
                  pl.BlockSpec((F, 256), lambda i: (0, 0)),
                  pl.BlockSpec((1, 256), lambda i: (0, 0)),
                  pl.BlockSpec((256, 256), lambda i: (0, 0)),
                  pl.BlockSpec((1, 256), lambda i: (0, 0)),
                  pl.BlockSpec((256, LATENT * NCH), lambda i: (0, 0)),
                  pl.BlockSpec((1, LATENT * NCH), lambda i: (0, 0))],
        out_specs=pl.BlockSpec((NGRAPH, LATENT * NCH), lambda i: (0, 0)),
        out_shape=jax.ShapeDtypeStruct((NGRAPH, LATENT * NCH), f32),
    )(s1, s2, h1_w, h1_b.reshape(1, 256), h2_w, h2_b.reshape(1, 256),
      h3_w, h3_b.reshape(1, LATENT * NCH))

    return out.reshape(NGRAPH, NCH, LATENT)
